# baseline (speedup 1.0000x reference)
import functools

import jax
import jax.numpy as jnp
from jax import lax
from jax.experimental import pallas as pl
from jax.experimental.pallas import tpu as pltpu
from jax.experimental.pallas import tpu_sc as plsc

_NC = 2
_NS = 16
_NW = _NC * _NS

_D = 8
_IV = 128
_SPC = 16
_CH = _IV * _SPC


def _body(n, b, xt_hbm, xf_hbm, perm_hbm, out_hbm, y_hbm, idx_v, rows_v, sem):
    wid = lax.axis_index("s") * _NC + lax.axis_index("c")
    per_w = n // _NW
    base_r = wid * (per_w // _IV)

    def chunk(s, c):
        off_r = base_r + s * _SPC
        pltpu.sync_copy(perm_hbm.at[pl.ds(off_r, _SPC)], idx_v)
        cps = [pltpu.async_copy(xt_hbm.at[idx_v.at[i]], rows_v.at[i], sem)
               for i in range(_SPC)]
        for cp in cps:
            cp.wait()
        pltpu.sync_copy(rows_v, out_hbm.at[pl.ds(off_r, _SPC)])
        return c

    lax.fori_loop(0, per_w // _CH, chunk, 0)

    cy = n // _NW
    for bb in range(b):
        pltpu.sync_copy(xf_hbm.at[bb].at[pl.ds(wid * cy, cy)],
                        y_hbm.at[bb].at[pl.ds(wid * cy, cy)])


@jax.jit
def _interleave(xt, xf, perm2):
    n, d = xt.shape
    b = xf.shape[0]
    mesh = plsc.VectorSubcoreMesh(core_axis_name="c", subcore_axis_name="s")
    k = pl.kernel(
        functools.partial(_body, n, b),
        out_type=(
            jax.ShapeDtypeStruct((n // _IV, _IV, d), jnp.float32),
            jax.ShapeDtypeStruct((b, n), jnp.float32),
        ),
        mesh=mesh,
        scratch_types=[
            pltpu.VMEM((_SPC, _IV), jnp.int32),
            pltpu.VMEM((_SPC, _IV, d), jnp.float32),
            pltpu.SemaphoreType.DMA,
        ],
        compiler_params=pltpu.CompilerParams(use_tc_tiling_on_sc=False),
    )
    return k(xt, xf, perm2)


def kernel(x, perm):
    bsz = x.shape[0]
    n = perm.shape[0]
    flat = x.reshape(bsz, n)
    xt = jnp.pad(flat.T, ((0, 0), (0, _D - bsz)))
    out_t, y = _interleave(xt, flat, perm.reshape(n // _IV, _IV))
    x_perm = out_t.reshape(n, _D)[:, :bsz].T.reshape(x.shape)
    return (x_perm, y.reshape(x.shape))


# speedup vs baseline: 9.7055x; 9.7055x over previous
import functools

import jax
import jax.numpy as jnp
from jax import lax
from jax.experimental import pallas as pl
from jax.experimental.pallas import tpu as pltpu
from jax.experimental.pallas import tpu_sc as plsc

_NC = 2
_NS = 16
_NW = _NC * _NS

_IV = 128
_SPC = 8
_CH = _IV * _SPC

_YC = 16384


def _body(n, b, xf_hbm, perm_hbm, out_hbm, y_hbm, idx_v, rows_v, ybuf_v, sem,
          sem_yi, sem_yo):
    wid = lax.axis_index("s") * _NC + lax.axis_index("c")
    per_w = n // _NW
    base_r = wid * (per_w // _IV)

    def chunk(s, c):
        off_r = base_r + s * _SPC
        pltpu.sync_copy(perm_hbm.at[pl.ds(off_r, _SPC)], idx_v)
        cps = []
        for bb in range(b):
            for i in range(_SPC):
                cps.append(pltpu.async_copy(
                    xf_hbm.at[bb].at[idx_v.at[i]], rows_v.at[bb].at[i], sem))
        for cp in cps:
            cp.wait()
        for bb in range(b):
            pltpu.sync_copy(rows_v.at[bb], out_hbm.at[bb].at[pl.ds(off_r, _SPC)])
        return c

    lax.fori_loop(0, per_w // _CH, chunk, 0)

    cy = n // _NW
    nsub = cy // _YC
    nsteps = b * nsub

    def y_src(t):
        bb = t // nsub
        off = wid * cy + (t % nsub) * _YC
        return xf_hbm.at[bb].at[pl.ds(off, _YC)]

    def y_dst(t):
        bb = t // nsub
        off = wid * cy + (t % nsub) * _YC
        return y_hbm.at[bb].at[pl.ds(off, _YC)]

    pltpu.async_copy(y_src(0), ybuf_v.at[0], sem_yi)

    def ystep(t, c):
        p = t % 2
        pltpu.make_async_copy(y_src(t), ybuf_v.at[p], sem_yi).wait()

        @pl.when(t >= 1)
        def _():
            pltpu.make_async_copy(ybuf_v.at[1 - p], y_dst(t - 1), sem_yo).wait()

        @pl.when(t + 1 < nsteps)
        def _():
            pltpu.async_copy(y_src(t + 1), ybuf_v.at[1 - p], sem_yi)

        pltpu.async_copy(ybuf_v.at[p], y_dst(t), sem_yo)
        return c

    lax.fori_loop(0, nsteps, ystep, 0)
    pltpu.make_async_copy(ybuf_v.at[(nsteps - 1) % 2], y_dst(nsteps - 1),
                          sem_yo).wait()


@jax.jit
def _interleave(xf, perm2):
    b, n = xf.shape
    mesh = plsc.VectorSubcoreMesh(core_axis_name="c", subcore_axis_name="s")
    k = pl.kernel(
        functools.partial(_body, n, b),
        out_type=(
            jax.ShapeDtypeStruct((b, n // _IV, _IV), jnp.float32),
            jax.ShapeDtypeStruct((b, n), jnp.float32),
        ),
        mesh=mesh,
        scratch_types=[
            pltpu.VMEM((_SPC, _IV), jnp.int32),
            pltpu.VMEM((b, _SPC, _IV), jnp.float32),
            pltpu.VMEM((2, _YC), jnp.float32),
            pltpu.SemaphoreType.DMA,
            pltpu.SemaphoreType.DMA,
            pltpu.SemaphoreType.DMA,
        ],
        compiler_params=pltpu.CompilerParams(use_tc_tiling_on_sc=False),
    )
    return k(xf, perm2)


def kernel(x, perm):
    bsz = x.shape[0]
    n = perm.shape[0]
    out, y = _interleave(x.reshape(bsz, n), perm.reshape(n // _IV, _IV))
    return (out.reshape(x.shape), y.reshape(x.shape))


# speedup vs baseline: 12.6025x; 1.2985x over previous
import functools

import jax
import jax.numpy as jnp
from jax import lax
from jax.experimental import pallas as pl
from jax.experimental.pallas import tpu as pltpu
from jax.experimental.pallas import tpu_sc as plsc

_NC = 2
_NS = 16
_NW = _NC * _NS

_D = 8
_IV = 128
_SPC = 16
_CH = _IV * _SPC

_R = 4096


def _iota16():
    return lax.broadcasted_iota(jnp.int32, (16,), 0)


def _build_body(n, b, xf_hbm, xt_hbm, y_hbm, tin_v, tbuf_v, sem):
    wid = lax.axis_index("s") * _NC + lax.axis_index("c")
    per_w = n // _NW
    base = wid * per_w
    iota = _iota16()

    def blk(s, c):
        lo = base + s * _R
        cps = [pltpu.async_copy(xf_hbm.at[bb].at[pl.ds(lo, _R)], tin_v.at[bb], sem)
               for bb in range(b)]
        for cp in cps:
            cp.wait()

        def grp(g, c2):
            vbase = g * 16
            for bb in range(b):
                v = tin_v[bb, pl.ds(vbase, 16)]
                idx = (vbase + iota) * _D + bb
                plsc.store_scatter(tbuf_v, [idx], v)
            return c2

        lax.fori_loop(0, _R // 16, grp, 0)
        pltpu.sync_copy(tbuf_v, xt_hbm.at[pl.ds(lo * _D, _R * _D)])
        for bb in range(b):
            pltpu.sync_copy(tin_v.at[bb], y_hbm.at[bb].at[pl.ds(lo, _R)])
        return c

    lax.fori_loop(0, per_w // _R, blk, 0)


def _gather_body(n, b, xt_hbm, perm_hbm, out_hbm, idx_v, rows_v, obuf_v, sem):
    wid = lax.axis_index("s") * _NC + lax.axis_index("c")
    per_w = n // _NW
    base = wid * per_w
    base_r = wid * (per_w // _IV)
    iota = _iota16()

    def chunk(s, c):
        off_r = base_r + s * _SPC
        pltpu.sync_copy(perm_hbm.at[pl.ds(off_r, _SPC)], idx_v)
        cps = [pltpu.async_copy(xt_hbm.at[idx_v.at[i]],
                                rows_v.at[pl.ds(i * _IV, _IV)], sem)
               for i in range(_SPC)]
        for cp in cps:
            cp.wait()

        def grp(g, c2):
            rows = g * 16 + iota
            for bb in range(b):
                col = jnp.full((16,), bb, jnp.int32)
                v = plsc.load_gather(rows_v, [rows, col])
                obuf_v[bb, pl.ds(g * 16, 16)] = v
            return c2

        lax.fori_loop(0, _CH // 16, grp, 0)
        jlo = base + s * _CH
        for bb in range(b):
            pltpu.sync_copy(obuf_v.at[bb], out_hbm.at[bb].at[pl.ds(jlo, _CH)])
        return c

    lax.fori_loop(0, per_w // _CH, chunk, 0)


@jax.jit
def _interleave(xf, perm2):
    b, n = xf.shape
    mesh = plsc.VectorSubcoreMesh(core_axis_name="c", subcore_axis_name="s")
    build = pl.kernel(
        functools.partial(_build_body, n, b),
        out_type=(
            jax.ShapeDtypeStruct((n * _D,), jnp.float32),
            jax.ShapeDtypeStruct((b, n), jnp.float32),
        ),
        mesh=mesh,
        scratch_types=[
            pltpu.VMEM((b, _R), jnp.float32),
            pltpu.VMEM((_R * _D,), jnp.float32),
            pltpu.SemaphoreType.DMA,
        ],
        compiler_params=pltpu.CompilerParams(use_tc_tiling_on_sc=False, needs_layout_passes=False),
    )
    xt_flat, y = build(xf)
    xt = xt_flat.reshape(n, _D)
    gather = pl.kernel(
        functools.partial(_gather_body, n, b),
        out_type=jax.ShapeDtypeStruct((b, n), jnp.float32),
        mesh=mesh,
        scratch_types=[
            pltpu.VMEM((_SPC, _IV), jnp.int32),
            pltpu.VMEM((_CH, _D), jnp.float32),
            pltpu.VMEM((b, _CH), jnp.float32),
            pltpu.SemaphoreType.DMA,
        ],
        compiler_params=pltpu.CompilerParams(use_tc_tiling_on_sc=False, needs_layout_passes=False),
    )
    out = gather(xt, perm2)
    return out, y


def kernel(x, perm):
    bsz = x.shape[0]
    n = perm.shape[0]
    out, y = _interleave(x.reshape(bsz, n), perm.reshape(n // _IV, _IV))
    return (out.reshape(x.shape), y.reshape(x.shape))


# speedup vs baseline: 13.9932x; 1.1104x over previous
import functools

import jax
import jax.numpy as jnp
from jax import lax
from jax.experimental import pallas as pl
from jax.experimental.pallas import tpu as pltpu
from jax.experimental.pallas import tpu_sc as plsc

_NC = 2
_NS = 16
_NW = _NC * _NS

_D = 8
_IV = 128
_SPC = 32
_CH = _IV * _SPC
_H = _SPC // 2

_R = 4096
_RH = _R // 2


def _iota16():
    return lax.broadcasted_iota(jnp.int32, (16,), 0)


def _build_body(n, b, xf_hbm, xt_hbm, y_hbm, tin_v, tbuf_v, sem0, sem1):
    wid = lax.axis_index("s") * _NC + lax.axis_index("c")
    per_w = n // _NW
    base = wid * per_w
    iota = _iota16()

    def interleave_half(h):
        def grp(g, c2):
            vbase = h * _RH + g * 16
            for bb in range(b):
                v = tin_v[bb, pl.ds(vbase, 16)]
                idx = (vbase + iota) * _D + bb
                plsc.store_scatter(tbuf_v, [idx], v)
            return c2

        lax.fori_loop(0, _RH // 16, grp, 0)

    def blk(s, c):
        lo = base + s * _R
        cps0 = [pltpu.async_copy(xf_hbm.at[bb].at[pl.ds(lo, _RH)],
                                 tin_v.at[bb].at[pl.ds(0, _RH)], sem0)
                for bb in range(b)]
        cps1 = [pltpu.async_copy(xf_hbm.at[bb].at[pl.ds(lo + _RH, _RH)],
                                 tin_v.at[bb].at[pl.ds(_RH, _RH)], sem1)
                for bb in range(b)]
        for cp in cps0:
            cp.wait()
        interleave_half(0)
        for cp in cps1:
            cp.wait()
        interleave_half(1)
        pltpu.sync_copy(tbuf_v, xt_hbm.at[pl.ds(lo * _D, _R * _D)])
        for bb in range(b):
            pltpu.sync_copy(tin_v.at[bb], y_hbm.at[bb].at[pl.ds(lo, _R)])
        return c

    lax.fori_loop(0, per_w // _R, blk, 0)


def _gather_body(n, b, xt_hbm, perm_hbm, out_hbm, idx_v, rows_v, obuf_v,
                 sem0, sem1):
    wid = lax.axis_index("s") * _NC + lax.axis_index("c")
    per_w = n // _NW
    base = wid * per_w
    base_r = wid * (per_w // _IV)
    iota = _iota16()
    hlen = _H * _IV

    def deint_half(h):
        def grp(g, c2):
            r0 = h * hlen + g * 16
            rows = r0 + iota
            for bb in range(b):
                col = jnp.full((16,), bb, jnp.int32)
                v = plsc.load_gather(rows_v, [rows, col])
                obuf_v[bb, pl.ds(r0, 16)] = v
            return c2

        lax.fori_loop(0, hlen // 16, grp, 0)

    def chunk(s, c):
        off_r = base_r + s * _SPC
        pltpu.sync_copy(perm_hbm.at[pl.ds(off_r, _SPC)], idx_v)
        cps0 = [pltpu.async_copy(xt_hbm.at[idx_v.at[i]],
                                 rows_v.at[pl.ds(i * _IV, _IV)], sem0)
                for i in range(_H)]
        cps1 = [pltpu.async_copy(xt_hbm.at[idx_v.at[_H + i]],
                                 rows_v.at[pl.ds((_H + i) * _IV, _IV)], sem1)
                for i in range(_H)]
        for cp in cps0:
            cp.wait()
        deint_half(0)
        for cp in cps1:
            cp.wait()
        deint_half(1)
        jlo = base + s * _CH
        for bb in range(b):
            pltpu.sync_copy(obuf_v.at[bb], out_hbm.at[bb].at[pl.ds(jlo, _CH)])
        return c

    lax.fori_loop(0, per_w // _CH, chunk, 0)


@jax.jit
def _interleave(xf, perm2):
    b, n = xf.shape
    mesh = plsc.VectorSubcoreMesh(core_axis_name="c", subcore_axis_name="s")
    build = pl.kernel(
        functools.partial(_build_body, n, b),
        out_type=(
            jax.ShapeDtypeStruct((n * _D,), jnp.float32),
            jax.ShapeDtypeStruct((b, n), jnp.float32),
        ),
        mesh=mesh,
        scratch_types=[
            pltpu.VMEM((b, _R), jnp.float32),
            pltpu.VMEM((_R * _D,), jnp.float32),
            pltpu.SemaphoreType.DMA,
            pltpu.SemaphoreType.DMA,
        ],
        compiler_params=pltpu.CompilerParams(use_tc_tiling_on_sc=False,
                                             needs_layout_passes=False),
    )
    xt_flat, y = build(xf)
    xt = xt_flat.reshape(n, _D)
    gather = pl.kernel(
        functools.partial(_gather_body, n, b),
        out_type=jax.ShapeDtypeStruct((b, n), jnp.float32),
        mesh=mesh,
        scratch_types=[
            pltpu.VMEM((_SPC, _IV), jnp.int32),
            pltpu.VMEM((_CH, _D), jnp.float32),
            pltpu.VMEM((b, _CH), jnp.float32),
            pltpu.SemaphoreType.DMA,
            pltpu.SemaphoreType.DMA,
        ],
        compiler_params=pltpu.CompilerParams(use_tc_tiling_on_sc=False,
                                             needs_layout_passes=False),
    )
    out = gather(xt, perm2)
    return out, y


def kernel(x, perm):
    bsz = x.shape[0]
    n = perm.shape[0]
    out, y = _interleave(x.reshape(bsz, n), perm.reshape(n // _IV, _IV))
    return (out.reshape(x.shape), y.reshape(x.shape))
